# Initial kernel scaffold; baseline (speedup 1.0000x reference)
import jax
import jax.numpy as jnp
from jax.experimental import pallas as pl


def kernel(input_ids, is_node, node_features, edge_index, mapping, embed_weight):
    raise NotImplementedError("write your pallas kernel here")


# speedup vs baseline: 1.6006x; 1.6006x over previous
import functools

import jax
import jax.numpy as jnp
from jax import lax
from jax.experimental import pallas as pl
from jax.experimental.pallas import tpu as pltpu
from jax.experimental.pallas import tpu_sc as plsc

D_MODEL = 1024
N_IDS = 4 * 4096

_NC, _NS = 2, 16
_NW = _NC * _NS
_PER_W = N_IDS // _NW
_CHUNK = 32
_NCHUNK = _PER_W // _CHUNK


def _embed_body(table_hbm, idx_hbm, out_hbm, idx_v, rows0, rows1, sem0, sem1):
    wid = lax.axis_index("s") * _NC + lax.axis_index("c")
    base = wid * _PER_W
    pltpu.sync_copy(idx_hbm.at[pl.ds(base, _PER_W)], idx_v)

    bufs = (rows0, rows1)
    sems = (sem0, sem1)

    copies = [
        pltpu.async_copy(table_hbm.at[idx_v.at[pl.ds(0, _CHUNK)]], rows0, sem0),
        None,
    ]
    for c in range(_NCHUNK):
        cur = c % 2
        if c + 1 < _NCHUNK:
            copies[1 - cur] = pltpu.async_copy(
                table_hbm.at[idx_v.at[pl.ds((c + 1) * _CHUNK, _CHUNK)]],
                bufs[1 - cur],
                sems[1 - cur],
            )
        copies[cur].wait()
        pltpu.sync_copy(bufs[cur], out_hbm.at[pl.ds(base + c * _CHUNK, _CHUNK)])


@jax.jit
def _embed_lookup(table, ids):
    mesh = plsc.VectorSubcoreMesh(core_axis_name="c", subcore_axis_name="s")
    run = pl.kernel(
        _embed_body,
        mesh=mesh,
        out_type=jax.ShapeDtypeStruct((N_IDS, D_MODEL), jnp.float32),
        scratch_types=[
            pltpu.VMEM((_PER_W,), jnp.int32),
            pltpu.VMEM((_CHUNK, D_MODEL), jnp.float32),
            pltpu.VMEM((_CHUNK, D_MODEL), jnp.float32),
            pltpu.SemaphoreType.DMA,
            pltpu.SemaphoreType.DMA,
        ],
    )
    return run(table, ids)


def kernel(input_ids, is_node, node_features, edge_index, mapping, embed_weight):
    ids = input_ids.reshape(-1)
    out = _embed_lookup(embed_weight, ids)
    return out.reshape(input_ids.shape[0], input_ids.shape[1], D_MODEL)


# speedup vs baseline: 1.6096x; 1.0056x over previous
import functools

import jax
import jax.numpy as jnp
from jax import lax
from jax.experimental import pallas as pl
from jax.experimental.pallas import tpu as pltpu
from jax.experimental.pallas import tpu_sc as plsc

D_MODEL = 1024
N_IDS = 4 * 4096

_NC, _NS = 2, 16
_NW = _NC * _NS
_PER_W = N_IDS // _NW
_CHUNK = 32
_NCHUNK = _PER_W // _CHUNK


def _embed_body(
    table_hbm, idx_hbm, out_hbm, idx_v, rows0, rows1, gsem0, gsem1, ssem0, ssem1
):
    wid = lax.axis_index("s") * _NC + lax.axis_index("c")
    base = wid * _PER_W
    pltpu.sync_copy(idx_hbm.at[pl.ds(base, _PER_W)], idx_v)

    bufs = (rows0, rows1)
    gsems = (gsem0, gsem1)
    ssems = (ssem0, ssem1)

    gcp = [
        pltpu.async_copy(table_hbm.at[idx_v.at[pl.ds(0, _CHUNK)]], rows0, gsem0),
        None,
    ]
    scp = [None, None]
    for c in range(_NCHUNK):
        cur = c % 2
        nxt = 1 - cur
        if c + 1 < _NCHUNK:
            if scp[nxt] is not None:
                scp[nxt].wait()
            gcp[nxt] = pltpu.async_copy(
                table_hbm.at[idx_v.at[pl.ds((c + 1) * _CHUNK, _CHUNK)]],
                bufs[nxt],
                gsems[nxt],
            )
        gcp[cur].wait()
        scp[cur] = pltpu.async_copy(
            bufs[cur], out_hbm.at[pl.ds(base + c * _CHUNK, _CHUNK)], ssems[cur]
        )
    for s in scp:
        if s is not None:
            s.wait()


@jax.jit
def _embed_lookup(table, ids):
    mesh = plsc.VectorSubcoreMesh(core_axis_name="c", subcore_axis_name="s")
    run = pl.kernel(
        _embed_body,
        mesh=mesh,
        out_type=jax.ShapeDtypeStruct((N_IDS, D_MODEL), jnp.float32),
        scratch_types=[
            pltpu.VMEM((_PER_W,), jnp.int32),
            pltpu.VMEM((_CHUNK, D_MODEL), jnp.float32),
            pltpu.VMEM((_CHUNK, D_MODEL), jnp.float32),
            pltpu.SemaphoreType.DMA,
            pltpu.SemaphoreType.DMA,
            pltpu.SemaphoreType.DMA,
            pltpu.SemaphoreType.DMA,
        ],
    )
    return run(table, ids)


def kernel(input_ids, is_node, node_features, edge_index, mapping, embed_weight):
    ids = input_ids.reshape(-1)
    out = _embed_lookup(embed_weight, ids)
    return out.reshape(input_ids.shape[0], input_ids.shape[1], D_MODEL)
